# Initial kernel scaffold; baseline (speedup 1.0000x reference)
import jax
import jax.numpy as jnp
from jax.experimental import pallas as pl


def kernel(input, embed_transmit, embed_receive, embed_orbit):
    raise NotImplementedError("write your pallas kernel here")


# speedup vs baseline: 3.5456x; 3.5456x over previous
import functools
import math

import jax
import jax.numpy as jnp
from jax import lax
from jax.experimental import pallas as pl
from jax.experimental.pallas import tpu as pltpu
from jax.experimental.pallas import tpu_sc as plsc

EMBED_DIM = 64
DIM1 = EMBED_DIM // 3
NUM_SAR = 12
NUM_OPT = 2500
NUM_ROWS = NUM_SAR + NUM_OPT

NC, NS = 2, 16
NW = NC * NS
B = 4096 * 200
BPW = B // NW
CHUNK = 128
NCHUNK = BPW // CHUNK


def _table_body(t_ref, r_ref, o_ref, out_ref):
    R, C = NUM_ROWS, EMBED_DIM
    r = lax.broadcasted_iota(jnp.int32, (R, C), 0)
    c = lax.broadcasted_iota(jnp.int32, (R, C), 1)
    pos = (r - NUM_SAR).astype(jnp.float32)
    j = (c % 32).astype(jnp.float32)
    omega = jnp.exp(j * (-math.log(10000.0) / 32.0))
    ang = pos * omega
    sincos = jnp.where(c < 32, jnp.sin(ang), jnp.cos(ang))
    s = 11 - r
    sm4 = s % 4
    q = s // 4
    t0 = jnp.broadcast_to(t_ref[0:1, :], (R, C))
    t1 = jnp.broadcast_to(t_ref[1:2, :], (R, C))
    r0 = jnp.broadcast_to(r_ref[0:1, :], (R, C))
    r1 = jnp.broadcast_to(r_ref[1:2, :], (R, C))
    o0 = jnp.broadcast_to(o_ref[0:1, :], (R, C))
    o1 = jnp.broadcast_to(o_ref[1:2, :], (R, C))
    tv = jnp.where(sm4 < 2, t0, t1)
    rv = jnp.where((sm4 == 0) | (sm4 == 3), r0, r1)
    ov = jnp.where(q == 0, 0.5 * (o0 + o1), jnp.where(q == 1, o0, o1))
    sarv = jnp.where(c < DIM1, tv, jnp.where(c < 2 * DIM1, rv, ov))
    out_ref[...] = jnp.where(r < NUM_SAR, sarv, sincos)


def _build_table(embed_transmit, embed_receive, embed_orbit):
    f32 = jnp.float32
    t = jnp.zeros((2, EMBED_DIM), f32).at[:, 0:DIM1].set(embed_transmit)
    r = jnp.zeros((2, EMBED_DIM), f32).at[:, DIM1:2 * DIM1].set(embed_receive)
    o = jnp.zeros((2, EMBED_DIM), f32).at[:, 2 * DIM1:].set(embed_orbit)
    return pl.pallas_call(
        _table_body,
        out_shape=jax.ShapeDtypeStruct((NUM_ROWS, EMBED_DIM), f32),
    )(t, r, o)


@functools.partial(
    pl.kernel,
    out_type=jax.ShapeDtypeStruct((B, EMBED_DIM), jnp.float32),
    mesh=plsc.VectorSubcoreMesh(core_axis_name="c", subcore_axis_name="s"),
    scratch_types=[
        pltpu.VMEM((CHUNK,), jnp.int32),
        pltpu.VMEM((CHUNK, EMBED_DIM), jnp.float32),
        pltpu.SemaphoreType.DMA,
    ],
    compiler_params=pltpu.CompilerParams(use_tc_tiling_on_sc=False),
)
def _gather(table_hbm, idx_hbm, out_hbm, idx_v, rows_v, sem):
    wid = lax.axis_index("s") * NC + lax.axis_index("c")
    base = wid * BPW

    def chunk_body(t, carry):
        start = base + t * CHUNK
        pltpu.sync_copy(idx_hbm.at[pl.ds(start, CHUNK)], idx_v)
        for i in range(CHUNK // 16):
            sl = pl.ds(i * 16, 16)
            idx_v[sl] = idx_v[sl] + NUM_SAR
        pltpu.async_copy(table_hbm.at[idx_v], rows_v, sem).wait()
        pltpu.sync_copy(rows_v, out_hbm.at[pl.ds(start, CHUNK)])
        return carry

    lax.fori_loop(0, NCHUNK, chunk_body, 0)


def kernel(input, embed_transmit, embed_receive, embed_orbit):
    table = _build_table(embed_transmit, embed_receive, embed_orbit)
    idx = input.reshape(-1).astype(jnp.int32)
    out = _gather(table, idx)
    return out.reshape(input.shape + (EMBED_DIM,))


# speedup vs baseline: 4.5303x; 1.2777x over previous
import functools
import math

import jax
import jax.numpy as jnp
from jax import lax
from jax.experimental import pallas as pl
from jax.experimental.pallas import tpu as pltpu
from jax.experimental.pallas import tpu_sc as plsc

EMBED_DIM = 64
DIM1 = EMBED_DIM // 3
NUM_SAR = 12
NUM_OPT = 2500
NUM_ROWS = NUM_SAR + NUM_OPT

NC, NS = 2, 16
NW = NC * NS
B = 4096 * 200
BPW = B // NW
GATHER = 128
CHUNK = 512
NGATHER = CHUNK // GATHER
NCHUNK = BPW // CHUNK


def _table_body(t_ref, r_ref, o_ref, out_ref):
    R, C = NUM_ROWS, EMBED_DIM
    r = lax.broadcasted_iota(jnp.int32, (R, C), 0)
    c = lax.broadcasted_iota(jnp.int32, (R, C), 1)
    pos = (r - NUM_SAR).astype(jnp.float32)
    j = (c % 32).astype(jnp.float32)
    omega = jnp.exp(j * (-math.log(10000.0) / 32.0))
    ang = pos * omega
    sincos = jnp.where(c < 32, jnp.sin(ang), jnp.cos(ang))
    s = 11 - r
    sm4 = s % 4
    q = s // 4
    t0 = jnp.broadcast_to(t_ref[0:1, :], (R, C))
    t1 = jnp.broadcast_to(t_ref[1:2, :], (R, C))
    r0 = jnp.broadcast_to(r_ref[0:1, :], (R, C))
    r1 = jnp.broadcast_to(r_ref[1:2, :], (R, C))
    o0 = jnp.broadcast_to(o_ref[0:1, :], (R, C))
    o1 = jnp.broadcast_to(o_ref[1:2, :], (R, C))
    tv = jnp.where(sm4 < 2, t0, t1)
    rv = jnp.where((sm4 == 0) | (sm4 == 3), r0, r1)
    ov = jnp.where(q == 0, 0.5 * (o0 + o1), jnp.where(q == 1, o0, o1))
    sarv = jnp.where(c < DIM1, tv, jnp.where(c < 2 * DIM1, rv, ov))
    out_ref[...] = jnp.where(r < NUM_SAR, sarv, sincos)


def _build_table(embed_transmit, embed_receive, embed_orbit):
    f32 = jnp.float32
    t = jnp.zeros((2, EMBED_DIM), f32).at[:, 0:DIM1].set(embed_transmit)
    r = jnp.zeros((2, EMBED_DIM), f32).at[:, DIM1:2 * DIM1].set(embed_receive)
    o = jnp.zeros((2, EMBED_DIM), f32).at[:, 2 * DIM1:].set(embed_orbit)
    return pl.pallas_call(
        _table_body,
        out_shape=jax.ShapeDtypeStruct((NUM_ROWS, EMBED_DIM), f32),
    )(t, r, o)


@functools.partial(
    pl.kernel,
    out_type=jax.ShapeDtypeStruct((B, EMBED_DIM), jnp.float32),
    mesh=plsc.VectorSubcoreMesh(core_axis_name="c", subcore_axis_name="s"),
    scratch_types=[
        pltpu.VMEM((BPW,), jnp.int32),
        pltpu.VMEM((2, CHUNK, EMBED_DIM), jnp.float32),
        pltpu.SemaphoreType.DMA,
        pltpu.SemaphoreType.DMA,
        pltpu.SemaphoreType.DMA,
        pltpu.SemaphoreType.DMA,
    ],
    compiler_params=pltpu.CompilerParams(use_tc_tiling_on_sc=False),
)
def _gather(table_hbm, idx_hbm, out_hbm, idx_v, rows_v,
            sem_g0, sem_g1, sem_o0, sem_o1):
    wid = lax.axis_index("s") * NC + lax.axis_index("c")
    base = wid * BPW
    sem_g = (sem_g0, sem_g1)
    sem_o = (sem_o0, sem_o1)

    pltpu.sync_copy(idx_hbm.at[pl.ds(base, BPW)], idx_v)

    def body(i, carry):
        for b in range(2):
            c = 2 * i + b
            off = c * CHUNK
            for k in range(CHUNK // 16):
                sl = pl.ds(off + k * 16, 16)
                idx_v[sl] = idx_v[sl] + NUM_SAR

            @pl.when(i > 0)
            def _():
                pltpu.make_async_copy(
                    rows_v.at[b], out_hbm.at[pl.ds(base + off, CHUNK)],
                    sem_o[b],
                ).wait()

            copies = [
                pltpu.async_copy(
                    table_hbm.at[idx_v.at[pl.ds(off + j * GATHER, GATHER)]],
                    rows_v.at[b, pl.ds(j * GATHER, GATHER)],
                    sem_g[b],
                )
                for j in range(NGATHER)
            ]
            for cp in copies:
                cp.wait()

            pltpu.async_copy(
                rows_v.at[b], out_hbm.at[pl.ds(base + off, CHUNK)], sem_o[b]
            )
        return carry

    lax.fori_loop(0, NCHUNK // 2, body, 0)

    last0 = (NCHUNK - 2) * CHUNK
    last1 = (NCHUNK - 1) * CHUNK
    pltpu.make_async_copy(
        rows_v.at[0], out_hbm.at[pl.ds(base + last0, CHUNK)], sem_o[0]
    ).wait()
    pltpu.make_async_copy(
        rows_v.at[1], out_hbm.at[pl.ds(base + last1, CHUNK)], sem_o[1]
    ).wait()


def kernel(input, embed_transmit, embed_receive, embed_orbit):
    table = _build_table(embed_transmit, embed_receive, embed_orbit)
    idx = input.reshape(-1).astype(jnp.int32)
    out = _gather(table, idx)
    return out.reshape(input.shape + (EMBED_DIM,))
